# Initial kernel scaffold; baseline (speedup 1.0000x reference)
import jax
import jax.numpy as jnp
from jax.experimental import pallas as pl


def kernel(xCellCenters, xFace, cf_w, fp_w, pp1_w, pl12_w, pp2_w, pl23_w, pp3_w, pl34_w, pp4_w, pc_w, W_cf, b_cf, W_fp, b_fp, W_pp1, b_pp1, W2a, b2a, W2b, b2b, W3a, b3a, W3b, b3b, W4a, b4a, W4b, b4b, W4c, b4c, W4d, b4d, W7a, b7a, W7b, b7b, W8a, b8a, W8b, b8b, W9a, b9a, W9b, b9b, Wf, bf, cf_src, cf_dst, fp_src, fp_dst, pp1_src, pp1_dst, pl12_idx, pp2_src, pp2_dst, pl23_idx, pp3_src, pp3_dst, pl34_idx, pp4_src, pp4_dst, pc_src, pc_dst):
    raise NotImplementedError("write your pallas kernel here")


# speedup vs baseline: 4.6153x; 4.6153x over previous
import functools

import jax
import jax.numpy as jnp
from jax import lax
from jax.experimental import pallas as pl
from jax.experimental.pallas import tpu as pltpu
from jax.experimental.pallas import tpu_sc as plsc

NC = 50000
NF = 100000
N1 = 100000
N2 = 25000
N3 = 6250
N4 = 1600

_NCORE = 2
_NSUB = 16
_NW = _NCORE * _NSUB
_CH = 128
_EPAD = _NW * _CH

_BN = 4096


def _cdiv(a, b):
    return -(-a // b)


def _pad_rows(a, n):
    if a.shape[0] == n:
        return a
    return jnp.pad(a, ((0, n - a.shape[0]),) + ((0, 0),) * (a.ndim - 1))


@functools.lru_cache(maxsize=None)
def _seg_accum_kernel(e_pad, nd_pad, f):
    per_w = e_pad // _NW
    nch = per_w // _CH
    rpt = nd_pad // _NSUB
    nf16 = f // 16
    mesh = plsc.VectorSubcoreMesh(core_axis_name="c", subcore_axis_name="s", num_cores=_NCORE, num_subcores=_NSUB)

    @functools.partial(
        pl.kernel,
        mesh=mesh,
        compiler_params=pltpu.CompilerParams(use_tc_tiling_on_sc=False),
        out_type=jax.ShapeDtypeStruct((2, nd_pad, f), jnp.float32),
        scratch_types=[
            pltpu.MemorySpace.VMEM_SHARED((nd_pad, f), jnp.float32),
            pltpu.MemorySpace.VMEM((_CH,), jnp.int32),
            pltpu.MemorySpace.VMEM((_CH,), jnp.int32),
            pltpu.MemorySpace.VMEM((_CH,), jnp.float32),
            pltpu.MemorySpace.VMEM((_CH, f), jnp.float32),
            pltpu.SemaphoreType.DMA,
        ],
    )
    def k(x_hbm, src_hbm, dst_hbm, w_hbm, zeros_hbm, out_hbm,
          acc, srcb, dstb, wb, rows, sem):
        cid = lax.axis_index("c")
        sid = lax.axis_index("s")
        wid = cid * _NSUB + sid

        r0 = sid * rpt
        pltpu.sync_copy(zeros_hbm.at[pl.ds(r0, rpt)], acc.at[pl.ds(r0, rpt)])
        plsc.subcore_barrier()

        ebase = wid * per_w

        def chunk(ci, carry):
            base = ebase + ci * _CH
            pltpu.sync_copy(src_hbm.at[pl.ds(base, _CH)], srcb)
            pltpu.sync_copy(w_hbm.at[pl.ds(base, _CH)], wb)
            pltpu.sync_copy(dst_hbm.at[pl.ds(base, _CH)], dstb)
            pltpu.async_copy(x_hbm.at[srcb], rows, sem).wait()

            def gbody(g, c2):
                wvec = wb[pl.ds(g * 16, 16)]
                for j in range(16):
                    wsp = wvec.at[jnp.full((16,), j, jnp.int32)].get(
                        mode="promise_in_bounds")
                    e = g * 16 + j
                    for f0 in range(nf16):
                        r = rows[e, pl.ds(f0 * 16, 16)]
                        rows[e, pl.ds(f0 * 16, 16)] = r * wsp
                return c2

            lax.fori_loop(0, _CH // 16, gbody, 0)
            pltpu.sync_copy(rows, acc.at[dstb], add=True)
            return carry

        lax.fori_loop(0, nch, chunk, 0)
        plsc.subcore_barrier()
        pltpu.sync_copy(acc.at[pl.ds(r0, rpt)],
                        out_hbm.at[cid, pl.ds(r0, rpt)])

    return k


def _seg(x, src, dst, w, nd):
    e = dst.shape[0]
    f = x.shape[1]
    e_pad = _cdiv(e, _EPAD) * _EPAD
    nd_pad = _cdiv(nd, 128) * 128
    if src is None:
        src = jnp.arange(e, dtype=jnp.int32)
    srcp = _pad_rows(src.astype(jnp.int32), e_pad)
    dstp = _pad_rows(dst.astype(jnp.int32), e_pad)
    wp = _pad_rows(w, e_pad)
    zeros = jnp.zeros((nd_pad, f), jnp.float32)
    k = _seg_accum_kernel(e_pad, nd_pad, f)
    return k(x, srcp, dstp, wp, zeros)


@functools.lru_cache(maxsize=None)
def _unpool_kernel(n_pad, nsrc, f):
    per_w = n_pad // _NW
    nch = per_w // _CH
    nf16 = f // 16
    mesh = plsc.VectorSubcoreMesh(core_axis_name="c", subcore_axis_name="s", num_cores=_NCORE, num_subcores=_NSUB)

    @functools.partial(
        pl.kernel,
        mesh=mesh,
        compiler_params=pltpu.CompilerParams(use_tc_tiling_on_sc=False),
        out_type=jax.ShapeDtypeStruct((n_pad, f), jnp.float32),
        scratch_types=[
            pltpu.MemorySpace.VMEM((_CH,), jnp.int32),
            pltpu.MemorySpace.VMEM((_CH,), jnp.float32),
            pltpu.MemorySpace.VMEM((_CH, f), jnp.float32),
            pltpu.SemaphoreType.DMA,
        ],
    )
    def k(xc_hbm, idx_hbm, pw_hbm, out_hbm, idxb, pwb, rows, sem):
        cid = lax.axis_index("c")
        sid = lax.axis_index("s")
        wid = cid * _NSUB + sid
        base0 = wid * per_w

        def chunk(ci, carry):
            base = base0 + ci * _CH
            pltpu.sync_copy(idx_hbm.at[pl.ds(base, _CH)], idxb)
            pltpu.sync_copy(pw_hbm.at[pl.ds(base, _CH)], pwb)
            pltpu.async_copy(xc_hbm.at[idxb], rows, sem).wait()

            def gbody(g, c2):
                wvec = pwb[pl.ds(g * 16, 16)]
                for j in range(16):
                    wsp = wvec.at[jnp.full((16,), j, jnp.int32)].get(
                        mode="promise_in_bounds")
                    e = g * 16 + j
                    for f0 in range(nf16):
                        r = rows[e, pl.ds(f0 * 16, 16)]
                        rows[e, pl.ds(f0 * 16, 16)] = r * wsp
                return c2

            lax.fori_loop(0, _CH // 16, gbody, 0)
            pltpu.sync_copy(rows, out_hbm.at[pl.ds(base, _CH)])
            return carry

        lax.fori_loop(0, nch, chunk, 0)

    return k


def _unpool(xc, idx, pw, n_out):
    f = xc.shape[1]
    n_pad = _cdiv(n_out, _EPAD) * _EPAD
    idxp = _pad_rows(idx.astype(jnp.int32), n_pad)
    pwp = _pad_rows(pw, n_pad)
    k = _unpool_kernel(n_pad, xc.shape[0], f)
    return k(xc, idxp, pwp)[:n_out]


def _tc_inputs(parts, fin):
    inputs, specs, meta = [], [], []
    for p in parts:
        inputs.append(p)
        if p.ndim == 3:
            specs.append(pl.BlockSpec((2, _BN, fin), lambda i: (0, i, 0)))
            meta.append(3)
        else:
            specs.append(pl.BlockSpec((_BN, fin), lambda i: (i, 0)))
            meta.append(2)
    return inputs, specs, meta


def _assemble(refs, meta, b_ref, relu):
    x = None
    for r, m in zip(refs, meta):
        v = (r[0] + r[1]) if m == 3 else r[...]
        x = v if x is None else x + v
    if b_ref is not None:
        x = x + b_ref[...]
    if relu:
        x = jnp.maximum(x, 0.0)
    return x


def _lin(parts, W0, b_out, relu_out, n_out, cv=(), post=(), prec=None):
    fin = parts[0].shape[-1]
    fout = W0.shape[1] if W0 is not None else fin
    grid = _cdiv(n_out, _BN)
    inputs, specs, meta = _tc_inputs(parts, fin)
    nb = len(inputs)
    if W0 is not None:
        inputs.append(W0)
        specs.append(pl.BlockSpec(W0.shape, lambda i: (0, 0)))
    cv_meta = []
    for cparts, V in cv:
        ci, cs, cm = _tc_inputs(cparts, cparts[0].shape[-1])
        inputs.extend(ci)
        specs.extend(cs)
        cv_meta.append(cm)
        inputs.append(V)
        specs.append(pl.BlockSpec(V.shape, lambda i: (0, 0)))
    if b_out is not None:
        inputs.append(b_out.reshape(1, fout))
        specs.append(pl.BlockSpec((1, fout), lambda i: (0, 0)))
    pi, ps, pm = _tc_inputs(post, fout) if post else ([], [], [])
    inputs.extend(pi)
    specs.extend(ps)

    def body(*refs):
        o = refs[-1]
        rs = list(refs[:-1])
        k = nb
        x = _assemble(rs[:nb], meta, None, False)
        if W0 is not None:
            x = jnp.dot(x, rs[k][...], preferred_element_type=jnp.float32,
                        precision=prec)
            k += 1
        for cm in cv_meta:
            c = _assemble(rs[k:k + len(cm)], cm, None, False)
            k += len(cm)
            x = x + jnp.dot(c, rs[k][...], preferred_element_type=jnp.float32,
                            precision=prec)
            k += 1
        if b_out is not None:
            x = x + rs[k][...]
            k += 1
        if relu_out:
            x = jnp.maximum(x, 0.0)
        if post:
            x = x + _assemble(rs[k:k + len(pm)], pm, None, False)
        o[...] = x

    return pl.pallas_call(
        body,
        grid=(grid,),
        in_specs=specs,
        out_specs=pl.BlockSpec((_BN, fout), lambda i: (i, 0)),
        out_shape=jax.ShapeDtypeStruct((n_out, fout), jnp.float32),
    )(*inputs)


def _stats(parts, b_in, relu, n_real, mean=None):
    fin = parts[0].shape[-1]
    grid = _cdiv(n_real, _BN)
    inputs, specs, meta = _tc_inputs(parts, fin)
    nb = len(inputs)
    if b_in is not None:
        inputs.append(b_in.reshape(1, fin))
        specs.append(pl.BlockSpec((1, fin), lambda i: (0, 0)))
    if mean is not None:
        inputs.append(mean)
        specs.append(pl.BlockSpec((1, fin), lambda i: (0, 0)))

    def body(*refs):
        o = refs[-1]
        rs = list(refs[:-1])
        k = nb
        b_ref = None
        if b_in is not None:
            b_ref = rs[k]
            k += 1
        x = _assemble(rs[:nb], meta, b_ref, relu)
        if mean is not None:
            x = x - rs[k][...] / n_real
        i = pl.program_id(0)
        rid = lax.broadcasted_iota(jnp.int32, x.shape, 0) + i * _BN
        xm = jnp.where(rid < n_real, x, 0.0)
        if mean is not None:
            xm = xm * xm
        st = jnp.sum(xm, axis=0, keepdims=True)

        @pl.when(i == 0)
        def _():
            o[...] = jnp.zeros_like(o)

        o[...] += st

    return pl.pallas_call(
        body,
        grid=(grid,),
        in_specs=specs,
        out_specs=pl.BlockSpec((1, fin), lambda i: (0, 0)),
        out_shape=jax.ShapeDtypeStruct((1, fin), jnp.float32),
    )(*inputs)


def _normalize(parts, b_in, relu, ssum, ssq, n_real, n_out):
    fin = parts[0].shape[-1]
    grid = _cdiv(n_out, _BN)
    inputs, specs, meta = _tc_inputs(parts, fin)
    nb = len(inputs)
    if b_in is not None:
        inputs.append(b_in.reshape(1, fin))
        specs.append(pl.BlockSpec((1, fin), lambda i: (0, 0)))
    inputs.append(ssum)
    specs.append(pl.BlockSpec((1, fin), lambda i: (0, 0)))
    inputs.append(ssq)
    specs.append(pl.BlockSpec((1, fin), lambda i: (0, 0)))

    def body(*refs):
        o = refs[-1]
        rs = list(refs[:-1])
        k = nb
        b_ref = None
        if b_in is not None:
            b_ref = rs[k]
            k += 1
        x = _assemble(rs[:nb], meta, b_ref, relu)
        m = rs[k][...] / n_real
        var = rs[k + 1][...] / n_real
        o[...] = (x - m) / jnp.sqrt(var + 1e-5)

    return pl.pallas_call(
        body,
        grid=(grid,),
        in_specs=specs,
        out_specs=pl.BlockSpec((_BN, fin), lambda i: (i, 0)),
        out_shape=jax.ShapeDtypeStruct((n_out, fin), jnp.float32),
    )(*inputs)


def _inorm(parts, b_in, relu, n_real):
    ssum = _stats(parts, b_in, relu, n_real)
    ssq = _stats(parts, b_in, relu, n_real, mean=ssum)
    return _normalize(parts, b_in, relu, ssum, ssq, n_real, n_real)


def kernel(xCellCenters, xFace, cf_w, fp_w, pp1_w, pl12_w, pp2_w, pl23_w, pp3_w, pl34_w, pp4_w, pc_w, W_cf, b_cf, W_fp, b_fp, W_pp1, b_pp1, W2a, b2a, W2b, b2b, W3a, b3a, W3b, b3b, W4a, b4a, W4b, b4b, W4c, b4c, W4d, b4d, W7a, b7a, W7b, b7b, W8a, b8a, W8b, b8b, W9a, b9a, W9b, b9b, Wf, bf, cf_src, cf_dst, fp_src, fp_dst, pp1_src, pp1_dst, pl12_idx, pp2_src, pp2_dst, pl23_idx, pp3_src, pp3_dst, pl34_idx, pp4_src, pp4_dst, pc_src, pc_dst):
    f32 = jnp.float32

    x0 = jnp.pad(xCellCenters[0], ((0, 0), (0, 14)))
    xf0 = jnp.pad(xFace[0], ((0, 0), (12, 0)))
    W_cf_p = jnp.zeros((16, 16), f32).at[:2, :12].set(W_cf)
    b_cf_p = jnp.pad(b_cf, (0, 4))

    xCn = _inorm([x0], None, False, NC)
    xFn = _inorm([xf0], None, False, NF)

    a_cf = _seg(xCn, cf_src, cf_dst, cf_w, NF)
    hcat1 = _lin([a_cf], W_cf_p, b_cf_p, True, NF, post=[xFn],
                 prec=lax.Precision.HIGHEST)
    a_fp = _seg(hcat1, fp_src, fp_dst, fp_w, N1)
    h1 = _lin([a_fp], W_fp, b_fp, True, N1)
    a1 = _seg(h1, pp1_src, pp1_dst, pp1_w, N1)
    x1 = _lin([h1, a1], W_pp1, b_pp1, True, N1)
    x1n = _inorm([x1], None, False, N1)

    p2 = _lin([_seg(x1n, None, pl12_idx, pl12_w, N2)], None, None, False, N2)
    a2a = _seg(p2, pp2_src, pp2_dst, pp2_w, N2)
    h2a = _lin([p2, a2a], W2a, b2a, True, N2)
    a2b = _seg(h2a, pp2_src, pp2_dst, pp2_w, N2)
    x2 = _lin([h2a, a2b], W2b, b2b, True, N2)
    x2n = _inorm([x2], None, False, N2)

    p3 = _lin([_seg(x2n, None, pl23_idx, pl23_w, N3)], None, None, False, N3)
    a3a = _seg(p3, pp3_src, pp3_dst, pp3_w, N3)
    h3a = _lin([p3, a3a], W3a, b3a, True, N3)
    a3b = _seg(h3a, pp3_src, pp3_dst, pp3_w, N3)
    x3 = _lin([h3a, a3b], W3b, b3b, True, N3)
    x3n = _inorm([x3], None, False, N3)

    p4 = _lin([_seg(x3n, None, pl34_idx, pl34_w, N4)], None, None, False, N4)
    a4a = _seg(p4, pp4_src, pp4_dst, pp4_w, N4)
    h4a = _lin([p4, a4a], W4a, b4a, True, N4)
    a4b = _seg(h4a, pp4_src, pp4_dst, pp4_w, N4)
    h4b = _lin([h4a, a4b], W4b, b4b, True, N4)
    a4c = _seg(h4b, pp4_src, pp4_dst, pp4_w, N4)
    h4c = _lin([h4b, a4c], W4c, b4c, True, N4)
    a4d = _seg(h4c, pp4_src, pp4_dst, pp4_w, N4)
    h4d = _lin([h4c, a4d], W4d, b4d, True, N4)

    x4 = _unpool(h4d, pl34_idx, pl34_w, N3)
    x4n = _inorm([x4], None, False, N3)
    hcat7 = jnp.concatenate([x4n, x3n], axis=1)
    a7a = _seg(hcat7, pp3_src, pp3_dst, pp3_w, N3)
    h7a = _lin([hcat7, a7a], W7a, b7a, True, N3)
    a7b = _seg(h7a, pp3_src, pp3_dst, pp3_w, N3)
    h7b = _lin([h7a, a7b], W7b, b7b, True, N3)

    x7 = _unpool(h7b, pl23_idx, pl23_w, N2)
    x7n = _inorm([x7], None, False, N2)
    hcat8 = jnp.concatenate([x7n, x2n], axis=1)
    a8a = _seg(hcat8, pp2_src, pp2_dst, pp2_w, N2)
    h8a = _lin([hcat8, a8a], W8a, b8a, True, N2)
    a8b = _seg(h8a, pp2_src, pp2_dst, pp2_w, N2)
    h8b = _lin([h8a, a8b], W8b, b8b, True, N2)

    x8 = _unpool(h8b, pl12_idx, pl12_w, N1)
    x8n = _inorm([x8], None, False, N1)
    a9a = _seg(x8n, pp1_src, pp1_dst, pp1_w, N1)
    a9b = _seg(x1n, pp1_src, pp1_dst, pp1_w, N1)
    h9 = _lin([x8n, a9a], W9a[:16], b9a, True, N1,
              cv=[([x1n, a9b], W9a[16:])])
    a_pc = _seg(h9, pc_src, pc_dst, pc_w, NC)
    h_pc = _lin([a_pc], W9b, b9b, True, NC)
    return _lin([h_pc], Wf, bf, False, NC)
